# Initial kernel scaffold; baseline (speedup 1.0000x reference)
import jax
import jax.numpy as jnp
from jax.experimental import pallas as pl


def kernel(x, node_embeddings1, T_i_D_emb, D_i_W_emb):
    raise NotImplementedError("write your pallas kernel here")


# speedup vs baseline: 2.9279x; 2.9279x over previous
import functools

import jax
import jax.numpy as jnp
from jax import lax
from jax.experimental import pallas as pl
from jax.experimental.pallas import tpu as pltpu
from jax.experimental.pallas import tpu_sc as plsc

_B = 4
_N = 10000
_T = 12
_D = 64
_TIMES = 288
_DAYS = 7

_C = 80
_NCHUNK = _N // _C
_ITEMS = _NCHUNK * _B
_NW = 32

_mesh = lambda: plsc.VectorSubcoreMesh(core_axis_name="c", subcore_axis_name="s")


_TDROWS = 2048
_TDCOLS = 128


@functools.partial(
    pl.kernel,
    out_type=jax.ShapeDtypeStruct((_TDROWS, _TDCOLS), jnp.float32),
    mesh=_mesh(),
    scratch_types=[
        pltpu.VMEM((_TIMES, _D), jnp.float32),
        pltpu.VMEM((_DAYS, _D), jnp.float32),
        pltpu.VMEM((64, _TDCOLS), jnp.float32),
    ],
)
def _td_build(t_hbm, d_hbm, out_hbm, tbuf, dbuf, obuf):
    wid = lax.axis_index("s") * 2 + lax.axis_index("c")
    pltpu.sync_copy(t_hbm, tbuf)
    pltpu.sync_copy(d_hbm, dbuf)
    for rl in range(64):
        r = wid * 64 + rl
        tr = jnp.minimum(r // _DAYS, _TIMES - 1)
        dr = jnp.minimum(r - _DAYS * (r // _DAYS), _DAYS - 1)
        for c in range(_TDCOLS // 16):
            sl = pl.ds(16 * c, 16)
            if c < _D // 16:
                obuf[rl, sl] = tbuf[tr, sl] * dbuf[dr, sl]
            else:
                obuf[rl, sl] = jnp.zeros((16,), jnp.float32)
    pltpu.sync_copy(obuf, out_hbm.at[pl.ds(wid * 64, 64)])


@functools.partial(
    pl.kernel,
    out_type=jax.ShapeDtypeStruct((_B * _T * _N, _D), jnp.float32),
    mesh=_mesh(),
    scratch_types=[
        pltpu.VMEM((2 * _T * _C,), jnp.float32),
        pltpu.VMEM((_C,), jnp.int32),
        pltpu.VMEM((_C, _TDCOLS), jnp.float32),
        pltpu.VMEM((_C, _D), jnp.float32),
        pltpu.VMEM((_C, _D), jnp.float32),
        pltpu.SemaphoreType.DMA,
        pltpu.SemaphoreType.DMA,
    ],
)
def _emb_lookup(xblk, node, td, out, xbuf, tidx, trows, nrows, orows,
                gsem, nsem):
    wid = lax.axis_index("s") * 2 + lax.axis_index("c")
    nitems = (_ITEMS + _NW - 1 - wid) // _NW

    def item_body(k, carry):
        i = wid + _NW * k
        chunk = i // _B
        b = i - _B * chunk
        n0 = chunk * _C
        ncopy = pltpu.async_copy(node.at[pl.ds(n0, _C)], nrows, nsem)
        pltpu.sync_copy(xblk.at[pl.ds((b * _NCHUNK + chunk) * 2 * _T * _C,
                                      2 * _T * _C)], xbuf)
        ncopy.wait()

        def t_body(t, carry2):
            for j in range(_C // 16):
                v1 = xbuf[pl.ds(t * _C + 16 * j, 16)]
                v2 = xbuf[pl.ds(_T * _C + t * _C + 16 * j, 16)]
                cidx = (v1 * float(_TIMES)).astype(jnp.int32) * _DAYS \
                    + v2.astype(jnp.int32)
                tidx[pl.ds(16 * j, 16)] = cidx
            pltpu.async_copy(td.at[tidx], trows, gsem).wait()

            def r_body(rb, c3):
                for u in range(4):
                    r = rb * 4 + u
                    for c in range(_D // 16):
                        sl = pl.ds(16 * c, 16)
                        orows[r, sl] = trows[r, sl] * nrows[r, sl]
                return c3

            lax.fori_loop(0, _C // 4, r_body, 0)
            bt = b * _T + t
            pltpu.sync_copy(orows, out.at[pl.ds(bt * _N + n0, _C)])
            return carry2

        lax.fori_loop(0, _T, t_body, 0)
        return carry

    lax.fori_loop(0, nitems, item_body, 0)


def kernel(x, node_embeddings1, T_i_D_emb, D_i_W_emb):
    xblk = jnp.transpose(
        x[:, :, 1:3, :].reshape(_B, _NCHUNK, _C, 2, _T),
        (0, 1, 3, 4, 2),
    ).reshape(-1)
    td = _td_build(T_i_D_emb, D_i_W_emb)
    flat = _emb_lookup(xblk, node_embeddings1, td)
    return flat.reshape(_B, _T, _N, _D), node_embeddings1


# speedup vs baseline: 3.1037x; 1.0600x over previous
import functools

import jax
import jax.numpy as jnp
from jax import lax
from jax.experimental import pallas as pl
from jax.experimental.pallas import tpu as pltpu
from jax.experimental.pallas import tpu_sc as plsc

_B = 4
_N = 10000
_T = 12
_D = 64
_TIMES = 288
_DAYS = 7

_C = 80
_NCHUNK = _N // _C
_ITEMS = _NCHUNK * _B
_NW = 32

_mesh = lambda: plsc.VectorSubcoreMesh(core_axis_name="c", subcore_axis_name="s")


_TDROWS = 2048
_TDCOLS = 128


@functools.partial(
    pl.kernel,
    out_type=jax.ShapeDtypeStruct((_TDROWS, _TDCOLS), jnp.float32),
    mesh=_mesh(),
    scratch_types=[
        pltpu.VMEM((_TIMES, _D), jnp.float32),
        pltpu.VMEM((_DAYS, _D), jnp.float32),
        pltpu.VMEM((64, _TDCOLS), jnp.float32),
    ],
)
def _td_build(t_hbm, d_hbm, out_hbm, tbuf, dbuf, obuf):
    wid = lax.axis_index("s") * 2 + lax.axis_index("c")
    pltpu.sync_copy(t_hbm, tbuf)
    pltpu.sync_copy(d_hbm, dbuf)
    for rl in range(64):
        r = wid * 64 + rl
        tr = jnp.minimum(r // _DAYS, _TIMES - 1)
        dr = jnp.minimum(r - _DAYS * (r // _DAYS), _DAYS - 1)
        for c in range(_TDCOLS // 16):
            sl = pl.ds(16 * c, 16)
            if c < _D // 16:
                obuf[rl, sl] = tbuf[tr, sl] * dbuf[dr, sl]
            else:
                obuf[rl, sl] = jnp.zeros((16,), jnp.float32)
    pltpu.sync_copy(obuf, out_hbm.at[pl.ds(wid * 64, 64)])


@functools.partial(
    pl.kernel,
    out_type=jax.ShapeDtypeStruct((_B * _T * _N, _D), jnp.float32),
    mesh=_mesh(),
    scratch_types=[
        pltpu.VMEM((2 * _T * _C,), jnp.float32),
        pltpu.VMEM((_C,), jnp.int32),
        pltpu.VMEM((_C,), jnp.int32),
        pltpu.VMEM((_C, _TDCOLS), jnp.float32),
        pltpu.VMEM((_C, _TDCOLS), jnp.float32),
        pltpu.VMEM((_C, _D), jnp.float32),
        pltpu.VMEM((_C, _D), jnp.float32),
        pltpu.VMEM((_C, _D), jnp.float32),
        pltpu.SemaphoreType.DMA,
        pltpu.SemaphoreType.DMA,
        pltpu.SemaphoreType.DMA,
    ],
)
def _emb_lookup(xblk, node, td, out, xbuf, tixa, tixb, tra, trb, ora, orb,
                nrows, gsem, nsem, wsem):
    wid = lax.axis_index("s") * 2 + lax.axis_index("c")
    nitems = (_ITEMS + _NW - 1 - wid) // _NW
    tix = (tixa, tixb)
    tr = (tra, trb)
    orw = (ora, orb)

    def compute_idx(t, dst):
        for j in range(_C // 16):
            v1 = xbuf[pl.ds(t * _C + 16 * j, 16)]
            v2 = xbuf[pl.ds(_T * _C + t * _C + 16 * j, 16)]
            dst[pl.ds(16 * j, 16)] = (v1 * float(_TIMES)).astype(jnp.int32) \
                * _DAYS + v2.astype(jnp.int32)

    def item_body(k, carry):
        i = wid + _NW * k
        chunk = i // _B
        b = i - _B * chunk
        n0 = chunk * _C
        ncopy = pltpu.async_copy(node.at[pl.ds(n0, _C)], nrows, nsem)
        pltpu.sync_copy(xblk.at[pl.ds((b * _NCHUNK + chunk) * 2 * _T * _C,
                                      2 * _T * _C)], xbuf)
        compute_idx(0, tix[0])
        gathers = [pltpu.async_copy(td.at[tix[0]], tr[0], gsem)]
        ncopy.wait()
        writes = []
        for t in range(_T):
            cur = t % 2
            nxt = (t + 1) % 2
            if t + 1 < _T:
                compute_idx(t + 1, tix[nxt])
                gathers.append(pltpu.async_copy(td.at[tix[nxt]], tr[nxt],
                                                gsem))
            gathers[t].wait()
            if t >= 2:
                writes[t - 2].wait()

            def r_body(rb, c3):
                for u in range(4):
                    r = rb * 4 + u
                    for c in range(_D // 16):
                        sl = pl.ds(16 * c, 16)
                        orw[cur][r, sl] = tr[cur][r, sl] * nrows[r, sl]
                return c3

            lax.fori_loop(0, _C // 4, r_body, 0)
            bt = b * _T + t
            writes.append(pltpu.async_copy(orw[cur],
                                           out.at[pl.ds(bt * _N + n0, _C)],
                                           wsem))
        writes[_T - 2].wait()
        writes[_T - 1].wait()
        return carry

    lax.fori_loop(0, nitems, item_body, 0)


def kernel(x, node_embeddings1, T_i_D_emb, D_i_W_emb):
    xblk = jnp.transpose(
        x[:, :, 1:3, :].reshape(_B, _NCHUNK, _C, 2, _T),
        (0, 1, 3, 4, 2),
    ).reshape(-1)
    td = _td_build(T_i_D_emb, D_i_W_emb)
    flat = _emb_lookup(xblk, node_embeddings1, td)
    return flat.reshape(_B, _T, _N, _D), node_embeddings1
